# baseline (speedup 1.0000x reference)
import functools

import jax
import jax.numpy as jnp
from jax import lax
from jax.experimental import pallas as pl
from jax.experimental.pallas import tpu as pltpu
from jax.experimental.pallas import tpu_sc as plsc

EMBED = 128
SEQ = 200
NC = 2
NS = 16
NW = NC * NS
IDXW = 40
NBUF = 5


@functools.partial(jax.jit, static_argnums=(3,))
def _tok_pos_embed(idx2d, token_table, pos_table, total):
    rows_per_w = total // NW
    chunks_per_w = rows_per_w // IDXW

    mesh = plsc.VectorSubcoreMesh(core_axis_name="c", subcore_axis_name="s")

    @functools.partial(
        pl.kernel,
        mesh=mesh,
        out_type=jax.ShapeDtypeStruct((total, EMBED), jnp.float32),
        scratch_types=[
            pltpu.VMEM((chunks_per_w, IDXW), jnp.int32),
            pltpu.VMEM((NBUF, IDXW, EMBED), jnp.float32),
            pltpu.VMEM_SHARED((SEQ, EMBED), jnp.float32),
            [pltpu.SemaphoreType.DMA] * NBUF,
            [pltpu.SemaphoreType.DMA] * NBUF,
        ],
    )
    def emb(idx_hbm, tok_hbm, pos_hbm, out_hbm, idx_v, rows_v, pos_v,
            gsems, osems):
        wid = lax.axis_index("s") * NC + lax.axis_index("c")

        @pl.when(lax.axis_index("s") == 0)
        def _():
            pltpu.sync_copy(pos_hbm, pos_v)

        plsc.subcore_barrier()
        pltpu.sync_copy(idx_hbm.at[pl.ds(wid * chunks_per_w, chunks_per_w)],
                        idx_v)

        def start_chunk(c, b):
            pltpu.sync_copy(pos_v.at[pl.ds((b % 5) * IDXW, IDXW)],
                            rows_v.at[b])
            pltpu.async_copy(tok_hbm.at[idx_v.at[c]], rows_v.at[b],
                             gsems[b], add=True)

        def finish_chunk(c, b):
            pltpu.make_async_copy(tok_hbm.at[idx_v.at[0]], rows_v.at[b],
                                  gsems[b]).wait()
            pltpu.async_copy(rows_v.at[b],
                             out_hbm.at[pl.ds(wid * rows_per_w + c * IDXW,
                                              IDXW)],
                             osems[b])

        def wait_out(b):
            pltpu.make_async_copy(
                rows_v.at[b],
                out_hbm.at[pl.ds(wid * rows_per_w, IDXW)],
                osems[b]).wait()

        def loop_body(g, carry):
            for b in range(NBUF):
                @pl.when(g > 0)
                def _():
                    wait_out(b)

                start_chunk(g * NBUF + b, b)
            for b in range(NBUF):
                finish_chunk(g * NBUF + b, b)
            return carry

        lax.fori_loop(0, chunks_per_w // NBUF, loop_body, 0)
        for b in range(NBUF):
            wait_out(b)

    return emb(idx2d, token_table, pos_table)


def kernel(x, token_table, pos_table):
    batch, seq = x.shape
    total = batch * seq
    idx2d = x.reshape(total // IDXW, IDXW).astype(jnp.int32)
    out = _tok_pos_embed(idx2d, token_table, pos_table, total)
    return out.reshape(batch, seq, EMBED)


# speedup vs baseline: 1.0538x; 1.0538x over previous
import functools

import jax
import jax.numpy as jnp
from jax import lax
from jax.experimental import pallas as pl
from jax.experimental.pallas import tpu as pltpu
from jax.experimental.pallas import tpu_sc as plsc

EMBED = 128
SEQ = 200
NC = 2
NS = 16
NW = NC * NS
IDXW = 100
CHUNK = 400
GPC = CHUNK // IDXW
NBUF = 2


@functools.partial(jax.jit, static_argnums=(3,))
def _tok_pos_embed(idx2d, token_table, pos_table, total):
    rows_per_w = total // NW
    chunks_per_w = rows_per_w // CHUNK

    mesh = plsc.VectorSubcoreMesh(core_axis_name="c", subcore_axis_name="s")

    @functools.partial(
        pl.kernel,
        mesh=mesh,
        out_type=jax.ShapeDtypeStruct((total, EMBED), jnp.float32),
        scratch_types=[
            pltpu.VMEM((NBUF, GPC, IDXW), jnp.int32),
            pltpu.VMEM((NBUF, CHUNK, EMBED), jnp.float32),
            pltpu.VMEM_SHARED((SEQ, EMBED), jnp.float32),
            [pltpu.SemaphoreType.DMA] * NBUF,
            [pltpu.SemaphoreType.DMA] * NBUF,
        ],
    )
    def emb(idx_hbm, tok_hbm, pos_hbm, out_hbm, idx_v, rows_v, pos_v,
            gsems, osems):
        wid = lax.axis_index("s") * NC + lax.axis_index("c")

        @pl.when(lax.axis_index("s") == 0)
        def _():
            pltpu.sync_copy(pos_hbm, pos_v)

        plsc.subcore_barrier()

        def start_chunk(c, b):
            pltpu.sync_copy(idx_hbm.at[pl.ds(wid * chunks_per_w * GPC
                                             + c * GPC, GPC)],
                            idx_v.at[b])
            for h in range(CHUNK // SEQ):
                pltpu.sync_copy(pos_v, rows_v.at[b].at[pl.ds(h * SEQ, SEQ)])
            for k in range(GPC):
                pltpu.async_copy(tok_hbm.at[idx_v.at[b].at[k]],
                                 rows_v.at[b].at[pl.ds(k * IDXW, IDXW)],
                                 gsems[b], add=True)

        def finish_chunk(c, b):
            for k in range(GPC):
                pltpu.make_async_copy(tok_hbm.at[idx_v.at[b].at[0]],
                                      rows_v.at[b].at[pl.ds(k * IDXW, IDXW)],
                                      gsems[b]).wait()
            pltpu.async_copy(rows_v.at[b],
                             out_hbm.at[pl.ds(wid * rows_per_w + c * CHUNK,
                                              CHUNK)],
                             osems[b])

        def wait_out(b):
            pltpu.make_async_copy(
                rows_v.at[b],
                out_hbm.at[pl.ds(wid * rows_per_w, CHUNK)],
                osems[b]).wait()

        def loop_body(g, carry):
            for b in range(NBUF):
                @pl.when(g > 0)
                def _():
                    wait_out(b)

                start_chunk(g * NBUF + b, b)
            for b in range(NBUF):
                finish_chunk(g * NBUF + b, b)
            return carry

        lax.fori_loop(0, chunks_per_w // NBUF, loop_body, 0)
        for b in range(NBUF):
            wait_out(b)

    return emb(idx2d, token_table, pos_table)


def kernel(x, token_table, pos_table):
    batch, seq = x.shape
    total = batch * seq
    idx2d = x.reshape(total // IDXW, IDXW).astype(jnp.int32)
    out = _tok_pos_embed(idx2d, token_table, pos_table, total)
    return out.reshape(batch, seq, EMBED)


# speedup vs baseline: 1.2132x; 1.1513x over previous
import functools

import jax
import jax.numpy as jnp
from jax import lax
from jax.experimental import pallas as pl
from jax.experimental.pallas import tpu as pltpu
from jax.experimental.pallas import tpu_sc as plsc

EMBED = 128
SEQ = 200
NC = 2
NS = 16
NW = NC * NS
IDXW = 100
NBUF = 3


@functools.partial(jax.jit, static_argnums=(3,))
def _tok_pos_embed(idx2d, token_table, pos_table, total):
    seq_per_w = total // NW // SEQ
    rows_per_w = total // NW
    irows_per_w = rows_per_w // IDXW

    mesh = plsc.VectorSubcoreMesh(core_axis_name="c", subcore_axis_name="s")

    @functools.partial(
        pl.kernel,
        mesh=mesh,
        out_type=jax.ShapeDtypeStruct((total, EMBED), jnp.float32),
        scratch_types=[
            pltpu.VMEM((irows_per_w, IDXW), jnp.int32),
            pltpu.VMEM((NBUF, SEQ, EMBED), jnp.float32),
            pltpu.VMEM_SHARED((SEQ, EMBED), jnp.float32),
            [pltpu.SemaphoreType.DMA] * NBUF,
            [pltpu.SemaphoreType.DMA] * NBUF,
        ],
    )
    def emb(idx_hbm, tok_hbm, pos_hbm, out_hbm, idx_v, rows_v, pos_v,
            gsems, osems):
        wid = lax.axis_index("s") * NC + lax.axis_index("c")

        @pl.when(lax.axis_index("s") == 0)
        def _():
            pltpu.sync_copy(pos_hbm, pos_v)

        plsc.subcore_barrier()
        pltpu.sync_copy(idx_hbm.at[pl.ds(wid * irows_per_w, irows_per_w)],
                        idx_v)

        def start_chunk(c, b):
            irow = c * (SEQ // IDXW)
            pltpu.sync_copy(pos_v, rows_v.at[b])
            pltpu.async_copy(tok_hbm.at[idx_v.at[irow]],
                             rows_v.at[b].at[pl.ds(0, IDXW)],
                             gsems[b], add=True)
            pltpu.async_copy(tok_hbm.at[idx_v.at[irow + 1]],
                             rows_v.at[b].at[pl.ds(IDXW, IDXW)],
                             gsems[b], add=True)

        def finish_chunk(c, b):
            pltpu.make_async_copy(tok_hbm.at[idx_v.at[0]],
                                  rows_v.at[b].at[pl.ds(0, IDXW)],
                                  gsems[b]).wait()
            pltpu.make_async_copy(tok_hbm.at[idx_v.at[0]],
                                  rows_v.at[b].at[pl.ds(IDXW, IDXW)],
                                  gsems[b]).wait()
            pltpu.async_copy(rows_v.at[b],
                             out_hbm.at[pl.ds(wid * rows_per_w + c * SEQ,
                                              SEQ)],
                             osems[b])

        def wait_out(b):
            pltpu.make_async_copy(
                rows_v.at[b],
                out_hbm.at[pl.ds(wid * rows_per_w, SEQ)],
                osems[b]).wait()

        ngroups = seq_per_w // NBUF
        ntail = seq_per_w - ngroups * NBUF

        def loop_body(g, carry):
            for b in range(NBUF):
                @pl.when(g > 0)
                def _():
                    wait_out(b)

                start_chunk(g * NBUF + b, b)
            for b in range(NBUF):
                finish_chunk(g * NBUF + b, b)
            return carry

        lax.fori_loop(0, ngroups, loop_body, 0)
        for b in range(ntail):
            wait_out(b)
            start_chunk(ngroups * NBUF + b, b)
        for b in range(ntail):
            finish_chunk(ngroups * NBUF + b, b)
        for b in range(NBUF):
            wait_out(b)

    return emb(idx2d, token_table, pos_table)


def kernel(x, token_table, pos_table):
    batch, seq = x.shape
    total = batch * seq
    idx2d = x.reshape(total // IDXW, IDXW).astype(jnp.int32)
    out = _tok_pos_embed(idx2d, token_table, pos_table, total)
    return out.reshape(batch, seq, EMBED)
